# baseline (speedup 1.0000x reference)
import functools

import jax
import jax.numpy as jnp
from jax.experimental import pallas as pl
from jax.experimental.pallas import tpu as pltpu

_T = 20
_I = 64
_H = 64
_E = 8
_G = 4
_GH = _G * _H
_F = 20


def _cv_sq(v_row, n):
    eps = 1e-10
    mean = jnp.sum(v_row, axis=1, keepdims=True) / n
    var = jnp.sum((v_row - mean) ** 2, axis=1, keepdims=True) / (n - 1)
    return var / (mean * mean + eps)


def _moe_body(x_ref, wg_ref, wih_ref, whh_ref, bih_ref, bhh_ref,
              fc1w_ref, fc1b_ref, fc2w_ref, fc2b_ref, y_ref, loss_ref,
              xha_ref, xhb_ref, wa_ref, wb_ref, fc1bd_ref):
    B = x_ref.shape[0]
    t = pl.program_id(0)

    @pl.when(t == 0)
    def _prologue():
        wa_ref[...] = jnp.zeros((_GH + _I, _GH), dtype=jnp.float32)
        wb_ref[...] = jnp.zeros((_GH + _I, _GH), dtype=jnp.float32)
        fc1bd_ref[...] = jnp.zeros((_E * _H, _E * _F), dtype=jnp.float32)
        for e in range(_G):
            lo, hi = e * _H, (e + 1) * _H
            wa_ref[lo:hi, lo:hi] = jnp.transpose(whh_ref[lo:hi, :])
            wa_ref[_GH:_GH + _I, lo:hi] = jnp.transpose(wih_ref[lo:hi, :])
            lo2, hi2 = (e + _G) * _H, (e + _G + 1) * _H
            wb_ref[lo:hi, lo:hi] = jnp.transpose(whh_ref[lo2:hi2, :])
            wb_ref[_GH:_GH + _I, lo:hi] = jnp.transpose(wih_ref[lo2:hi2, :])
        for e in range(_E):
            fc1bd_ref[e * _H:(e + 1) * _H, e * _F:(e + 1) * _F] = (
                jnp.transpose(fc1w_ref[e * _F:(e + 1) * _F, :]))
        xha_ref[:, 0:_GH] = jnp.zeros((B, _GH), dtype=jnp.float32)
        xhb_ref[:, 0:_GH] = jnp.zeros((B, _GH), dtype=jnp.float32)

    bcat = bih_ref[...] + bhh_ref[...]
    ba = bcat[:, 0:_GH]
    bb = bcat[:, _GH:2 * _GH]

    for s in range(2):
        xt = x_ref[:, s * _I:(s + 1) * _I]
        xha_ref[:, _GH:_GH + _I] = xt
        xhb_ref[:, _GH:_GH + _I] = xt
        pre_a = jnp.dot(xha_ref[...], wa_ref[...],
                        preferred_element_type=jnp.float32) + ba
        pre_b = jnp.dot(xhb_ref[...], wb_ref[...],
                        preferred_element_type=jnp.float32) + bb
        ha = jnp.tanh(pre_a)
        hb = jnp.tanh(pre_b)
        xha_ref[:, 0:_GH] = ha
        xhb_ref[:, 0:_GH] = hb

    @pl.when(t == _T // 2 - 1)
    def _epilogue():
        h = jnp.concatenate([ha, hb], axis=1)
        z = jnp.tanh(jnp.dot(h, fc1bd_ref[...],
                             preferred_element_type=jnp.float32)
                     + fc1b_ref[...])
        seg_r = jax.lax.broadcasted_iota(jnp.int32, (_E * _F, _E), 0) // _F
        seg_c = jax.lax.broadcasted_iota(jnp.int32, (_E * _F, _E), 1)
        seg = jnp.where(seg_r == seg_c, 1.0, 0.0)
        out = (jnp.dot(z * fc2w_ref[...], seg,
                       preferred_element_type=jnp.float32) + fc2b_ref[...])

        logits = jnp.dot(xt, wg_ref[...],
                         preferred_element_type=jnp.float32)
        iota = jax.lax.broadcasted_iota(jnp.int32, (B, _E), 1)
        m1 = jnp.max(logits, axis=1, keepdims=True)
        i1 = jnp.min(jnp.where(logits == m1, iota, _E), axis=1, keepdims=True)
        oh1 = (iota == i1)
        masked = jnp.where(oh1, -jnp.inf, logits)
        m2 = jnp.max(masked, axis=1, keepdims=True)
        i2 = jnp.min(jnp.where(masked == m2, iota, _E), axis=1, keepdims=True)
        oh2 = (iota == i2)
        e2 = jnp.exp(m2 - m1)
        denom = 1.0 + e2
        g1 = 1.0 / denom
        g2 = e2 / denom
        gates = jnp.where(oh1, g1, 0.0) + jnp.where(oh2, g2, 0.0)

        y_ref[...] = jnp.sum(gates * out, axis=1, keepdims=True)

        importance = jnp.sum(gates, axis=0, keepdims=True)
        load = jnp.sum(jnp.where(gates > 0, 1.0, 0.0), axis=0, keepdims=True)
        loss_ref[...] = (_cv_sq(importance, _E) + _cv_sq(load, _E)) * 1e-2


@functools.partial(jax.jit, static_argnames=())
def kernel(x, w_gate, W_ih, W_hh, b_ih, b_hh, fc1_w, fc1_b, fc2_w, fc2_b):
    B, T, I = x.shape
    E = w_gate.shape[1]
    H = W_ih.shape[1]
    F = fc1_w.shape[1]

    x2d = x.reshape(B, T * I)
    wih2d = W_ih.reshape(E * H, I)
    whh2d = W_hh.reshape(E * H, H)
    bih2d = b_ih.reshape(1, E * H)
    bhh2d = b_hh.reshape(1, E * H)
    fc1w2d = fc1_w.reshape(E * F, H)
    fc1b2d = fc1_b.reshape(1, E * F)
    fc2w2d = fc2_w.reshape(1, E * F)
    fc2b2d = fc2_b.reshape(1, E)

    full = lambda shape: pl.BlockSpec(shape, lambda t: (0, 0))
    y, loss = pl.pallas_call(
        _moe_body,
        grid=(T // 2,),
        in_specs=[
            pl.BlockSpec((B, 2 * I), lambda t: (0, t)),
            full((I, E)),
            full((E * H, I)),
            full((E * H, H)),
            full((1, E * H)),
            full((1, E * H)),
            full((E * F, H)),
            full((1, E * F)),
            full((1, E * F)),
            full((1, E)),
        ],
        out_specs=(
            full((B, 1)),
            full((1, 1)),
        ),
        out_shape=(
            jax.ShapeDtypeStruct((B, 1), jnp.float32),
            jax.ShapeDtypeStruct((1, 1), jnp.float32),
        ),
        scratch_shapes=[
            pltpu.VMEM((B, _GH + _I), jnp.float32),
            pltpu.VMEM((B, _GH + _I), jnp.float32),
            pltpu.VMEM((_GH + _I, _GH), jnp.float32),
            pltpu.VMEM((_GH + _I, _GH), jnp.float32),
            pltpu.VMEM((_E * _H, _E * _F), jnp.float32),
        ],
    )(x2d, w_gate, wih2d, whh2d, bih2d, bhh2d, fc1w2d, fc1b2d,
      fc2w2d, fc2b2d)
    return y, loss.reshape(())


# speedup vs baseline: 1.0279x; 1.0279x over previous
import functools

import jax
import jax.numpy as jnp
from jax.experimental import pallas as pl
from jax.experimental.pallas import tpu as pltpu

_T = 20
_I = 64
_H = 64
_E = 8
_G = 4
_GH = _G * _H
_F = 20


def _cv_sq(v_row, n):
    eps = 1e-10
    mean = jnp.sum(v_row, axis=1, keepdims=True) / n
    var = jnp.sum((v_row - mean) ** 2, axis=1, keepdims=True) / (n - 1)
    return var / (mean * mean + eps)


def _moe_body(x_ref, wg_ref, wih_ref, whh_ref, bih_ref, bhh_ref,
              fc1w_ref, fc1b_ref, fc2w_ref, fc2b_ref, y_ref, loss_ref,
              xha_ref, xhb_ref, wa_ref, wb_ref, fc1bd_ref):
    B = x_ref.shape[0]

    wa_ref[...] = jnp.zeros((_GH + _I, _GH), dtype=jnp.float32)
    wb_ref[...] = jnp.zeros((_GH + _I, _GH), dtype=jnp.float32)
    fc1bd_ref[...] = jnp.zeros((_E * _H, _E * _F), dtype=jnp.float32)
    for e in range(_G):
        lo, hi = e * _H, (e + 1) * _H
        wa_ref[lo:hi, lo:hi] = jnp.transpose(whh_ref[lo:hi, :])
        wa_ref[_GH:_GH + _I, lo:hi] = jnp.transpose(wih_ref[lo:hi, :])
        lo2, hi2 = (e + _G) * _H, (e + _G + 1) * _H
        wb_ref[lo:hi, lo:hi] = jnp.transpose(whh_ref[lo2:hi2, :])
        wb_ref[_GH:_GH + _I, lo:hi] = jnp.transpose(wih_ref[lo2:hi2, :])
    for e in range(_E):
        fc1bd_ref[e * _H:(e + 1) * _H, e * _F:(e + 1) * _F] = (
            jnp.transpose(fc1w_ref[e * _F:(e + 1) * _F, :]))

    bcat = bih_ref[...] + bhh_ref[...]
    ba = bcat[:, 0:_GH]
    bb = bcat[:, _GH:2 * _GH]
    wa = wa_ref[...]
    wb = wb_ref[...]

    xha_ref[:, 0:_GH] = jnp.zeros((B, _GH), dtype=jnp.float32)
    xhb_ref[:, 0:_GH] = jnp.zeros((B, _GH), dtype=jnp.float32)
    xt0 = x_ref[:, 0:_I]
    xha_ref[:, _GH:_GH + _I] = xt0
    xhb_ref[:, _GH:_GH + _I] = xt0
    for t in range(_T):
        pre_a = jnp.dot(xha_ref[...], wa,
                        preferred_element_type=jnp.float32) + ba
        pre_b = jnp.dot(xhb_ref[...], wb,
                        preferred_element_type=jnp.float32) + bb
        ha = jnp.tanh(pre_a)
        hb = jnp.tanh(pre_b)
        if t < _T - 1:
            xha_ref[:, 0:_GH] = ha
            xhb_ref[:, 0:_GH] = hb
            xt = x_ref[:, (t + 1) * _I:(t + 2) * _I]
            xha_ref[:, _GH:_GH + _I] = xt
            xhb_ref[:, _GH:_GH + _I] = xt

    h = jnp.concatenate([ha, hb], axis=1)
    z = jnp.tanh(jnp.dot(h, fc1bd_ref[...],
                         preferred_element_type=jnp.float32) + fc1b_ref[...])
    seg_r = jax.lax.broadcasted_iota(jnp.int32, (_E * _F, _E), 0) // _F
    seg_c = jax.lax.broadcasted_iota(jnp.int32, (_E * _F, _E), 1)
    seg = jnp.where(seg_r == seg_c, 1.0, 0.0)
    fc2_bd = jnp.transpose(fc2w_ref[...]) * seg
    out = (jnp.dot(z, fc2_bd,
                   preferred_element_type=jnp.float32) + fc2b_ref[...])

    logits = jnp.dot(x_ref[:, (_T - 1) * _I:_T * _I], wg_ref[...],
                     preferred_element_type=jnp.float32)
    iota = jax.lax.broadcasted_iota(jnp.int32, (B, _E), 1)
    m1 = jnp.max(logits, axis=1, keepdims=True)
    i1 = jnp.min(jnp.where(logits == m1, iota, _E), axis=1, keepdims=True)
    oh1 = (iota == i1)
    masked = jnp.where(oh1, -jnp.inf, logits)
    m2 = jnp.max(masked, axis=1, keepdims=True)
    i2 = jnp.min(jnp.where(masked == m2, iota, _E), axis=1, keepdims=True)
    oh2 = (iota == i2)
    e2 = jnp.exp(m2 - m1)
    denom = 1.0 + e2
    g1 = 1.0 / denom
    g2 = e2 / denom
    gates = jnp.where(oh1, g1, 0.0) + jnp.where(oh2, g2, 0.0)

    y_ref[...] = jnp.sum(gates * out, axis=1, keepdims=True)

    importance = jnp.sum(gates, axis=0, keepdims=True)
    load = jnp.sum(jnp.where(gates > 0, 1.0, 0.0), axis=0, keepdims=True)
    loss_ref[...] = (_cv_sq(importance, _E) + _cv_sq(load, _E)) * 1e-2


@functools.partial(jax.jit, static_argnames=())
def kernel(x, w_gate, W_ih, W_hh, b_ih, b_hh, fc1_w, fc1_b, fc2_w, fc2_b):
    B, T, I = x.shape
    E = w_gate.shape[1]
    H = W_ih.shape[1]
    F = fc1_w.shape[1]

    x2d = x.reshape(B, T * I)
    wih2d = W_ih.reshape(E * H, I)
    whh2d = W_hh.reshape(E * H, H)
    bih2d = b_ih.reshape(1, E * H)
    bhh2d = b_hh.reshape(1, E * H)
    fc1w2d = fc1_w.reshape(E * F, H)
    fc1b2d = fc1_b.reshape(1, E * F)
    fc2w2d = fc2_w.reshape(1, E * F)
    fc2b2d = fc2_b.reshape(1, E)

    y, loss = pl.pallas_call(
        _moe_body,
        out_shape=(
            jax.ShapeDtypeStruct((B, 1), jnp.float32),
            jax.ShapeDtypeStruct((1, 1), jnp.float32),
        ),
        scratch_shapes=[
            pltpu.VMEM((B, _GH + _I), jnp.float32),
            pltpu.VMEM((B, _GH + _I), jnp.float32),
            pltpu.VMEM((_GH + _I, _GH), jnp.float32),
            pltpu.VMEM((_GH + _I, _GH), jnp.float32),
            pltpu.VMEM((_E * _H, _E * _F), jnp.float32),
        ],
    )(x2d, w_gate, wih2d, whh2d, bih2d, bhh2d, fc1w2d, fc1b2d,
      fc2w2d, fc2b2d)
    return y, loss.reshape(())
